# Initial kernel scaffold; baseline (speedup 1.0000x reference)
import jax
import jax.numpy as jnp
from jax.experimental import pallas as pl


def kernel(inputs, emb_mat):
    raise NotImplementedError("write your pallas kernel here")


# speedup vs baseline: 4.5369x; 4.5369x over previous
import functools

import jax
import jax.numpy as jnp
from jax import lax
from jax.experimental import pallas as pl
from jax.experimental.pallas import tpu as pltpu
from jax.experimental.pallas import tpu_sc as plsc

VOCAB = 100000
EMB_DIM = 64
BATCH = 4096
SEQ = 50

NUM_WORKERS = 32
TOTAL = BATCH * SEQ
B_PER_W = TOTAL // NUM_WORKERS
CHUNK = 128
NCHUNK = B_PER_W // CHUNK


def _emb_kernel(idx_hbm, table_hbm, out_hbm, idx_v, rows_v, gsem, ssem):
    wid = lax.axis_index("s") * 2 + lax.axis_index("c")
    base = wid * B_PER_W

    pltpu.sync_copy(idx_hbm.at[pl.ds(base, B_PER_W)], idx_v)

    def start_gather(g, buf):
        pltpu.async_copy(
            table_hbm.at[idx_v.at[pl.ds(g * CHUNK, CHUNK)]],
            rows_v.at[buf],
            gsem,
        )

    def wait_gather(buf):
        pltpu.make_async_copy(
            table_hbm.at[idx_v.at[pl.ds(0, CHUNK)]], rows_v.at[buf], gsem
        ).wait()

    def start_store(g, buf):
        pltpu.async_copy(
            rows_v.at[buf], out_hbm.at[pl.ds(base + g * CHUNK, CHUNK)], ssem
        )

    def wait_store(g, buf):
        pltpu.make_async_copy(
            rows_v.at[buf], out_hbm.at[pl.ds(base + g * CHUNK, CHUNK)], ssem
        ).wait()

    start_gather(0, 0)

    def body(g, _):
        buf = lax.rem(g, 2)
        nbuf = 1 - buf
        start_gather(g + 1, nbuf)
        wait_gather(buf)
        start_store(g, buf)
        wait_store(g, buf)
        return 0

    lax.fori_loop(0, NCHUNK - 1, body, 0)

    last = NCHUNK - 1
    lbuf = last % 2
    wait_gather(lbuf)
    start_store(last, lbuf)
    wait_store(last, lbuf)


@jax.jit
def _embedding_lookup(idx_flat, emb_mat):
    mesh = plsc.VectorSubcoreMesh(core_axis_name="c", subcore_axis_name="s")
    f = pl.kernel(
        _emb_kernel,
        out_type=jax.ShapeDtypeStruct((TOTAL, EMB_DIM), jnp.float32),
        mesh=mesh,
        scratch_types=[
            pltpu.VMEM((B_PER_W,), jnp.int32),
            pltpu.VMEM((2, CHUNK, EMB_DIM), jnp.float32),
            pltpu.SemaphoreType.DMA,
            pltpu.SemaphoreType.DMA,
        ],
        compiler_params=pltpu.CompilerParams(use_tc_tiling_on_sc=False),
    )
    return f(idx_flat, emb_mat)


def kernel(inputs, emb_mat):
    idx_flat = inputs.reshape(TOTAL).astype(jnp.int32)
    out = _embedding_lookup(idx_flat, emb_mat)
    return out.reshape(BATCH, SEQ, EMB_DIM)


# speedup vs baseline: 4.6260x; 1.0196x over previous
import jax
import jax.numpy as jnp
from jax import lax
from jax.experimental import pallas as pl
from jax.experimental.pallas import tpu as pltpu
from jax.experimental.pallas import tpu_sc as plsc

VOCAB = 100000
EMB_DIM = 64
BATCH = 4096
SEQ = 50

NUM_WORKERS = 32
TOTAL = BATCH * SEQ
B_PER_W = TOTAL // NUM_WORKERS
CHUNK = 128
K = 5
GROUP = CHUNK * K
NGRP = B_PER_W // GROUP
NPAIR = NGRP // 2


def _emb_kernel(idx_hbm, table_hbm, out_hbm,
                idx_v, rows0, rows1, gsem0, gsem1, ssem0, ssem1):
    wid = lax.axis_index("s") * 2 + lax.axis_index("c")
    base = wid * B_PER_W

    pltpu.sync_copy(idx_hbm.at[pl.ds(base, B_PER_W)], idx_v)

    rows = (rows0, rows1)
    gsems = (gsem0, gsem1)
    ssems = (ssem0, ssem1)

    def gather_copy(grp, s, j):
        return pltpu.make_async_copy(
            table_hbm.at[idx_v.at[pl.ds(grp * GROUP + j * CHUNK, CHUNK)]],
            rows[s].at[j],
            gsems[s],
        )

    def store_copy(grp, s, j):
        return pltpu.make_async_copy(
            rows[s].at[j],
            out_hbm.at[pl.ds(base + grp * GROUP + j * CHUNK, CHUNK)],
            ssems[s],
        )

    def start_gathers(grp, s):
        for j in range(K):
            gather_copy(grp, s, j).start()

    def wait_gathers(grp, s):
        for j in range(K):
            gather_copy(grp, s, j).wait()

    def start_stores(grp, s):
        for j in range(K):
            store_copy(grp, s, j).start()

    def wait_stores(grp, s):
        for j in range(K):
            store_copy(grp, s, j).wait()

    start_gathers(0, 0)

    def body(p, _):
        grp0 = 2 * p
        grp1 = grp0 + 1
        wait_gathers(grp0, 0)

        @pl.when(p > 0)
        def _():
            wait_stores(grp0 - 1, 1)

        start_gathers(grp1, 1)
        start_stores(grp0, 0)
        wait_gathers(grp1, 1)
        wait_stores(grp0, 0)

        @pl.when(p < NPAIR - 1)
        def _():
            start_gathers(grp0 + 2, 0)

        start_stores(grp1, 1)
        return 0

    lax.fori_loop(0, NPAIR, body, 0)
    wait_stores(NGRP - 1, 1)


@jax.jit
def _embedding_lookup(idx_flat, emb_mat):
    mesh = plsc.VectorSubcoreMesh(core_axis_name="c", subcore_axis_name="s")
    f = pl.kernel(
        _emb_kernel,
        out_type=jax.ShapeDtypeStruct((TOTAL, EMB_DIM), jnp.float32),
        mesh=mesh,
        scratch_types=[
            pltpu.VMEM((B_PER_W,), jnp.int32),
            pltpu.VMEM((K, CHUNK, EMB_DIM), jnp.float32),
            pltpu.VMEM((K, CHUNK, EMB_DIM), jnp.float32),
            pltpu.SemaphoreType.DMA,
            pltpu.SemaphoreType.DMA,
            pltpu.SemaphoreType.DMA,
            pltpu.SemaphoreType.DMA,
        ],
        compiler_params=pltpu.CompilerParams(use_tc_tiling_on_sc=False),
    )
    return f(idx_flat, emb_mat)


def kernel(inputs, emb_mat):
    idx_flat = inputs.reshape(TOTAL).astype(jnp.int32)
    out = _embedding_lookup(idx_flat, emb_mat)
    return out.reshape(BATCH, SEQ, EMB_DIM)


# speedup vs baseline: 9.9391x; 2.1485x over previous
import jax
import jax.numpy as jnp
from jax import lax
from jax.experimental import pallas as pl
from jax.experimental.pallas import tpu as pltpu
from jax.experimental.pallas import tpu_sc as plsc

VOCAB = 100000
EMB_DIM = 64
BATCH = 4096
SEQ = 50

NUM_WORKERS = 32
D_PER_W = EMB_DIM // NUM_WORKERS
NPAIR = SEQ // 2


def _emb_kernel(idx_hbm, tab_hbm, out_hbm,
                rowbuf, ib0, ib1, ob0, ob1, isem0, isem1, osem0, osem1):
    wid = lax.axis_index("s") * 2 + lax.axis_index("c")

    ibufs = (ib0, ib1)
    obufs = (ob0, ob1)
    isems = (isem0, isem1)
    osems = (osem0, osem1)

    def icopy(s, b):
        return pltpu.make_async_copy(idx_hbm.at[s], ibufs[b], isems[b])

    def ocopy(s, d, b):
        return pltpu.make_async_copy(obufs[b], out_hbm.at[s, d], osems[b])

    def compute(b):
        @plsc.parallel_loop(0, BATCH, step=16, unroll=8)
        def _(k):
            idx16 = ibufs[b][pl.ds(k, 16)]
            obufs[b][pl.ds(k, 16)] = plsc.load_gather(rowbuf, [idx16])

    for rep in range(D_PER_W):
        d = wid + NUM_WORKERS * rep
        pltpu.sync_copy(tab_hbm.at[d], rowbuf)
        icopy(0, 0).start()
        icopy(1, 1).start()

        def body(p, _):
            for b in range(2):
                s = 2 * p + b
                icopy(s, b).wait()

                @pl.when(p > 0)
                def _():
                    ocopy(s - 2, d, b).wait()

                compute(b)
                ocopy(s, d, b).start()

                @pl.when(p < NPAIR - 1)
                def _():
                    icopy(s + 2, b).start()

            return 0

        lax.fori_loop(0, NPAIR, body, 0)
        ocopy(SEQ - 2, d, 0).wait()
        ocopy(SEQ - 1, d, 1).wait()


@jax.jit
def _embedding_lookup(idx_t, tab_t):
    mesh = plsc.VectorSubcoreMesh(core_axis_name="c", subcore_axis_name="s")
    f = pl.kernel(
        _emb_kernel,
        out_type=jax.ShapeDtypeStruct((SEQ, EMB_DIM, BATCH), jnp.float32),
        mesh=mesh,
        scratch_types=[
            pltpu.VMEM((VOCAB,), jnp.float32),
            pltpu.VMEM((BATCH,), jnp.int32),
            pltpu.VMEM((BATCH,), jnp.int32),
            pltpu.VMEM((BATCH,), jnp.float32),
            pltpu.VMEM((BATCH,), jnp.float32),
            pltpu.SemaphoreType.DMA,
            pltpu.SemaphoreType.DMA,
            pltpu.SemaphoreType.DMA,
            pltpu.SemaphoreType.DMA,
        ],
        compiler_params=pltpu.CompilerParams(needs_layout_passes=False),
    )
    return f(idx_t, tab_t)


def kernel(inputs, emb_mat):
    idx_t = jnp.transpose(inputs, (1, 0)).astype(jnp.int32)
    tab_t = jnp.transpose(emb_mat, (1, 0))
    out_t = _embedding_lookup(idx_t, tab_t)
    return jnp.transpose(out_t, (2, 0, 1))


# speedup vs baseline: 9.9548x; 1.0016x over previous
import jax
import jax.numpy as jnp
from jax import lax
from jax.experimental import pallas as pl
from jax.experimental.pallas import tpu as pltpu
from jax.experimental.pallas import tpu_sc as plsc

VOCAB = 100000
EMB_DIM = 64
BATCH = 4096
SEQ = 50

NUM_WORKERS = 32
D_PER_W = EMB_DIM // NUM_WORKERS
NPAIR = SEQ // 2


def _emb_kernel(idx_hbm, tab_hbm, out_hbm,
                rowbuf, ib0, ib1, ob0, ob1, isem0, isem1, osem0, osem1):
    wid = lax.axis_index("s") * 2 + lax.axis_index("c")

    ibufs = (ib0, ib1)
    obufs = (ob0, ob1)
    isems = (isem0, isem1)
    osems = (osem0, osem1)

    def icopy(s, b):
        return pltpu.make_async_copy(idx_hbm.at[s], ibufs[b], isems[b])

    def ocopy(s, d, b):
        return pltpu.make_async_copy(obufs[b], out_hbm.at[s, d], osems[b])

    def compute(b):
        @plsc.parallel_loop(0, BATCH, step=16, unroll=16)
        def _(k):
            idx16 = ibufs[b][pl.ds(k, 16)]
            obufs[b][pl.ds(k, 16)] = plsc.load_gather(rowbuf, [idx16])

    for rep in range(D_PER_W):
        d = wid + NUM_WORKERS * rep
        pltpu.sync_copy(tab_hbm.at[d], rowbuf)
        icopy(0, 0).start()
        icopy(1, 1).start()

        def body(p, _):
            for b in range(2):
                s = 2 * p + b
                icopy(s, b).wait()

                @pl.when(p > 0)
                def _():
                    ocopy(s - 2, d, b).wait()

                compute(b)
                ocopy(s, d, b).start()

                @pl.when(p < NPAIR - 1)
                def _():
                    icopy(s + 2, b).start()

            return 0

        lax.fori_loop(0, NPAIR, body, 0)
        ocopy(SEQ - 2, d, 0).wait()
        ocopy(SEQ - 1, d, 1).wait()


@jax.jit
def _embedding_lookup(idx_t, tab_t):
    mesh = plsc.VectorSubcoreMesh(core_axis_name="c", subcore_axis_name="s")
    f = pl.kernel(
        _emb_kernel,
        out_type=jax.ShapeDtypeStruct((SEQ, EMB_DIM, BATCH), jnp.float32),
        mesh=mesh,
        scratch_types=[
            pltpu.VMEM((VOCAB,), jnp.float32),
            pltpu.VMEM((BATCH,), jnp.int32),
            pltpu.VMEM((BATCH,), jnp.int32),
            pltpu.VMEM((BATCH,), jnp.float32),
            pltpu.VMEM((BATCH,), jnp.float32),
            pltpu.SemaphoreType.DMA,
            pltpu.SemaphoreType.DMA,
            pltpu.SemaphoreType.DMA,
            pltpu.SemaphoreType.DMA,
        ],
        compiler_params=pltpu.CompilerParams(needs_layout_passes=False),
    )
    return f(idx_t, tab_t)


def kernel(inputs, emb_mat):
    idx_t = jnp.transpose(inputs, (1, 0)).astype(jnp.int32)
    tab_t = jnp.transpose(emb_mat, (1, 0))
    out_t = _embedding_lookup(idx_t, tab_t)
    return jnp.transpose(out_t, (2, 0, 1))


# speedup vs baseline: 14.2972x; 1.4362x over previous
import jax
import jax.numpy as jnp
from jax import lax
from jax.experimental import pallas as pl
from jax.experimental.pallas import tpu as pltpu
from jax.experimental.pallas import tpu_sc as plsc

VOCAB = 100000
EMB_DIM = 64
BATCH = 4096
SEQ = 50

NUM_WORKERS = 32
D_PER_W = EMB_DIM // NUM_WORKERS


def _emb_kernel(idx_hbm, tab_hbm, out_hbm,
                idx_sp, rowbuf, ib0, ib1, ob0, ob1,
                isem0, isem1, osem0, osem1):
    cid = lax.axis_index("c")
    sid = lax.axis_index("s")
    wid = sid * 2 + cid

    @pl.when(sid == 0)
    def _():
        pltpu.sync_copy(idx_hbm, idx_sp)

    plsc.subcore_barrier()

    ibufs = (ib0, ib1)
    obufs = (ob0, ob1)
    isems = (isem0, isem1)
    osems = (osem0, osem1)

    def iread(s):
        return pltpu.make_async_copy(
            idx_sp.at[pl.ds(s * BATCH, BATCH)], ibufs[s % 2], isems[s % 2]
        )

    def ocopy(s, d):
        return pltpu.make_async_copy(
            obufs[s % 2], out_hbm.at[s, d], osems[s % 2]
        )

    def compute(s):
        ib = ibufs[s % 2]
        ob = obufs[s % 2]

        @plsc.parallel_loop(0, BATCH, step=16, unroll=4)
        def _(k):
            idx16 = ib[pl.ds(k, 16)]
            ob[pl.ds(k, 16)] = plsc.load_gather(rowbuf, [idx16])

    for rep in range(D_PER_W):
        d = wid + NUM_WORKERS * rep
        pltpu.sync_copy(tab_hbm.at[d], rowbuf)
        iread(0).start()
        for s in range(SEQ):
            iread(s).wait()
            if s + 1 < SEQ:
                iread(s + 1).start()
            if s >= 2 or rep > 0:
                ocopy(s % 2, 0).wait()
            compute(s)
            ocopy(s, d).start()
    ocopy(0, 0).wait()
    ocopy(1, 0).wait()


@jax.jit
def _embedding_lookup(idx_t, tab_t):
    mesh = plsc.VectorSubcoreMesh(core_axis_name="c", subcore_axis_name="s")
    f = pl.kernel(
        _emb_kernel,
        out_type=jax.ShapeDtypeStruct((SEQ, EMB_DIM, BATCH), jnp.float32),
        mesh=mesh,
        scratch_types=[
            pltpu.MemorySpace.VMEM_SHARED((SEQ * BATCH,), jnp.int32),
            pltpu.VMEM((VOCAB,), jnp.float32),
            pltpu.VMEM((BATCH,), jnp.int32),
            pltpu.VMEM((BATCH,), jnp.int32),
            pltpu.VMEM((BATCH,), jnp.float32),
            pltpu.VMEM((BATCH,), jnp.float32),
            pltpu.SemaphoreType.DMA,
            pltpu.SemaphoreType.DMA,
            pltpu.SemaphoreType.DMA,
            pltpu.SemaphoreType.DMA,
        ],
        compiler_params=pltpu.CompilerParams(needs_layout_passes=False),
    )
    return f(idx_t, tab_t)


def kernel(inputs, emb_mat):
    idx_t = jnp.transpose(inputs, (1, 0)).astype(jnp.int32)
    tab_t = jnp.transpose(emb_mat, (1, 0))
    out_t = _embedding_lookup(idx_t.reshape(SEQ * BATCH), tab_t)
    return jnp.transpose(out_t, (2, 0, 1))
